# baseline (speedup 1.0000x reference)
import functools

import jax
import jax.numpy as jnp
from jax import lax
from jax.experimental import pallas as pl
from jax.experimental.pallas import tpu as pltpu
from jax.experimental.pallas import tpu_sc as plsc

N = 10000
NP = 10240
D = 128
E_BASE = 160000
ES = E_BASE + N
NC = 2
NS = 16
NW = NC * NS
CHUNK = 32
CH_PER_W = 2 * (-(-ES // (NW * CHUNK * 2)))
EPW = CH_PER_W * CHUNK
ES_PAD = NW * EPW
ROWS_PER_TILE = NP // NS
BLK = 1024


def _sc_layer(xl, xr, src, dst, att):
    mesh = plsc.VectorSubcoreMesh(core_axis_name="c", subcore_axis_name="s")

    @functools.partial(
        pl.kernel,
        out_type=[jax.ShapeDtypeStruct((NC * NP, D), jnp.float32),
                  jax.ShapeDtypeStruct((NW, NP), jnp.float32)],
        mesh=mesh,
        scratch_types=[
            pltpu.VMEM((2, CHUNK), jnp.int32),
            pltpu.VMEM((2, CHUNK), jnp.int32),
            pltpu.VMEM((2, CHUNK, D), jnp.float32),
            pltpu.VMEM((2, CHUNK, D), jnp.float32),
            pltpu.VMEM((16, 32), jnp.float32),
            pltpu.VMEM((D,), jnp.float32),
            pltpu.VMEM((NP,), jnp.float32),
            pltpu.VMEM_SHARED((NP, D), jnp.float32),
            pltpu.SemaphoreType.DMA,
            pltpu.SemaphoreType.DMA,
            pltpu.SemaphoreType.DMA,
            pltpu.SemaphoreType.DMA,
        ],
    )
    def body(xl_hbm, xr_hbm, src_hbm, dst_hbm, att_hbm, num_hbm, den_hbm,
             src_v, dst_v, xlr, xrr, red, att_v, den_v, acc_sh,
             sem_l0, sem_r0, sem_l1, sem_r1):
        cid = lax.axis_index("c")
        sid = lax.axis_index("s")
        wid = cid * NS + sid
        pltpu.sync_copy(att_hbm, att_v)
        a_j = [att_v[pl.ds(16 * j, 16)] for j in range(D // 16)]

        lane = lax.iota(jnp.int32, 16)
        zero16 = jnp.zeros((16,), jnp.float32)
        zero16i = jnp.zeros((16,), jnp.int32)

        def zero_xlr(e, carry):
            for j in range(D // 16):
                xlr[0, e, pl.ds(j * 16, 16)] = zero16
                xlr[1, e, pl.ds(j * 16, 16)] = zero16
            return carry
        lax.fori_loop(0, CHUNK, zero_xlr, 0)

        def zero_den(t, carry):
            den_v[pl.ds(t * 16, 16)] = zero16
            return carry
        lax.fori_loop(0, NP // 16, zero_den, 0)

        base_row = sid * ROWS_PER_TILE
        for off in range(0, ROWS_PER_TILE, CHUNK):
            pltpu.sync_copy(xlr.at[0], acc_sh.at[pl.ds(base_row + off, CHUNK)])
        plsc.subcore_barrier()

        sems = ((sem_l0, sem_r0), (sem_l1, sem_r1))
        cbase = wid * CH_PER_W

        def issue(b, ci):
            ebase = (cbase + ci) * CHUNK
            pltpu.sync_copy(src_hbm.at[pl.ds(ebase, CHUNK)], src_v.at[b])
            pltpu.sync_copy(dst_hbm.at[pl.ds(ebase, CHUNK)], dst_v.at[b])
            pltpu.async_copy(xl_hbm.at[src_v.at[b]], xlr.at[b], sems[b][0])
            pltpu.async_copy(xr_hbm.at[dst_v.at[b]], xrr.at[b], sems[b][1])

        def wait(b):
            pltpu.make_async_copy(xl_hbm.at[src_v.at[b]], xlr.at[b],
                                  sems[b][0]).wait()
            pltpu.make_async_copy(xr_hbm.at[dst_v.at[b]], xrr.at[b],
                                  sems[b][1]).wait()

        def compute(b):
            for g in range(CHUNK // 16):
                dv16 = dst_v[b, pl.ds(g * 16, 16)]
                for l in range(16):
                    e = g * 16 + l
                    acc0 = zero16
                    acc1 = zero16
                    xs = []
                    for j in range(D // 16):
                        vl = xlr[b, e, pl.ds(16 * j, 16)]
                        vr = xrr[b, e, pl.ds(16 * j, 16)]
                        xs.append(vl)
                        z = vl + vr
                        z = jnp.maximum(z, 0.2 * z)
                        if j % 2 == 0:
                            acc0 = acc0 + z * a_j[j]
                        else:
                            acc1 = acc1 + z * a_j[j]
                    acc = acc0 + acc1
                    for sh in (8, 4, 2, 1):
                        red[l, pl.ds(0, 16)] = acc
                        red[l, pl.ds(16, 16)] = acc
                        acc = acc + red[l, pl.ds(sh, 16)]
                    pv = jnp.exp(acc)
                    for j in range(D // 16):
                        xlr[b, e, pl.ds(16 * j, 16)] = xs[j] * pv
                    dd = dv16[l]
                    dbase = (dd >> 4) << 4
                    m = dd & 15
                    diff = jnp.abs(lane - (zero16i + m))
                    oh = jnp.maximum(1 - diff, 0).astype(jnp.float32)
                    den_v[pl.ds(dbase, 16)] = (
                        den_v[pl.ds(dbase, 16)] + pv * oh)
            pltpu.sync_copy(xlr.at[b], acc_sh.at[dst_v.at[b]], add=True)

        last = CH_PER_W - 1
        issue(0, 0)

        def pair_body(i, carry):
            c0 = 2 * i
            issue(1, jnp.minimum(c0 + 1, last))
            wait(0)
            compute(0)
            issue(0, jnp.minimum(c0 + 2, last))
            wait(1)
            compute(1)
            return carry
        lax.fori_loop(0, CH_PER_W // 2, pair_body, 0)
        wait(0)

        plsc.subcore_barrier()
        pltpu.sync_copy(acc_sh.at[pl.ds(base_row, ROWS_PER_TILE)],
                        num_hbm.at[pl.ds(cid * NP + base_row, ROWS_PER_TILE)])
        pltpu.sync_copy(den_v, den_hbm.at[wid])

    return body(xl, xr, src, dst, att)


def _tc_first(x, wl, wr):
    def tc_body(x_ref, wl_ref, wr_ref, xl_ref, xr_ref):
        h = x_ref[...]
        xl_ref[...] = jnp.dot(h, wl_ref[...], preferred_element_type=jnp.float32)
        xr_ref[...] = jnp.dot(h, wr_ref[...], preferred_element_type=jnp.float32)

    return pl.pallas_call(
        tc_body,
        grid=(NP // BLK,),
        in_specs=[pl.BlockSpec((BLK, D), lambda i: (i, 0)),
                  pl.BlockSpec((D, D), lambda i: (0, 0)),
                  pl.BlockSpec((D, D), lambda i: (0, 0))],
        out_specs=[pl.BlockSpec((BLK, D), lambda i: (i, 0)),
                   pl.BlockSpec((BLK, D), lambda i: (i, 0))],
        out_shape=[jax.ShapeDtypeStruct((NP, D), jnp.float32)] * 2,
    )(x, wl, wr)


def _den_col(dens_blk, eye_blk):
    s = jnp.sum(dens_blk, axis=0, keepdims=True)
    sb = jnp.broadcast_to(s, (8, BLK))
    mm = lax.dot_general(eye_blk, sb, (((1,), (1,)), ((), ())),
                         preferred_element_type=jnp.float32)
    return mm[:, 0:1]


def _tc_mid(num, dens, eye, bias, wl, wr):
    def tc_body(num_ref, den_ref, eye_ref, b_ref, wl_ref, wr_ref,
                xl_ref, xr_ref):
        s = num_ref[0] + num_ref[1]
        dcol = _den_col(den_ref[...], eye_ref[...])
        h = s / (dcol + 1e-16) + b_ref[...]
        h = jnp.maximum(h, 0.0)
        xl_ref[...] = jnp.dot(h, wl_ref[...], preferred_element_type=jnp.float32)
        xr_ref[...] = jnp.dot(h, wr_ref[...], preferred_element_type=jnp.float32)

    return pl.pallas_call(
        tc_body,
        grid=(NP // BLK,),
        in_specs=[pl.BlockSpec((2, BLK, D), lambda i: (0, i, 0)),
                  pl.BlockSpec((NW, BLK), lambda i: (0, i)),
                  pl.BlockSpec((BLK, BLK), lambda i: (0, 0)),
                  pl.BlockSpec((1, D), lambda i: (0, 0)),
                  pl.BlockSpec((D, D), lambda i: (0, 0)),
                  pl.BlockSpec((D, D), lambda i: (0, 0))],
        out_specs=[pl.BlockSpec((BLK, D), lambda i: (i, 0)),
                   pl.BlockSpec((BLK, D), lambda i: (i, 0))],
        out_shape=[jax.ShapeDtypeStruct((NP, D), jnp.float32)] * 2,
    )(num, dens, eye, bias, wl, wr)


def _tc_tail(num, dens, eye):
    def tc_body(num_ref, den_ref, eye_ref, out_ref):
        i = pl.program_id(0)
        s = num_ref[0] + num_ref[1]
        dcol = _den_col(den_ref[...], eye_ref[...])
        h = s / (dcol + 1e-16)
        rows = i * BLK + lax.broadcasted_iota(jnp.int32, (BLK, 1), 0)
        h = jnp.where(rows < N, h, 0.0)
        part = jnp.sum(h, axis=0, keepdims=True)

        @pl.when(i == 0)
        def _():
            out_ref[...] = jnp.zeros_like(out_ref)

        out_ref[0:1, :] += part

    return pl.pallas_call(
        tc_body,
        grid=(NP // BLK,),
        in_specs=[pl.BlockSpec((2, BLK, D), lambda i: (0, i, 0)),
                  pl.BlockSpec((NW, BLK), lambda i: (0, i)),
                  pl.BlockSpec((BLK, BLK), lambda i: (0, 0))],
        out_specs=pl.BlockSpec((8, D), lambda i: (0, 0)),
        out_shape=jax.ShapeDtypeStruct((8, D), jnp.float32),
    )(num, dens, eye)


def _tc_head(s_sum, l_sum, r_sum, bs, bl, br, w1s, w1l, w1r, b1, w2, b2, w3, b3):
    inv_n = 1.0 / N

    def tc_body(ss_ref, sl_ref, sr_ref, bs_ref, bl_ref, br_ref,
                w1s_ref, w1l_ref, w1r_ref, b1_ref, w2_ref, b2_ref,
                w3_ref, b3_ref, out_ref):
        es = ss_ref[...] * inv_n + bs_ref[...]
        el = sl_ref[...] * inv_n + bl_ref[...]
        er = sr_ref[...] * inv_n + br_ref[...]
        h = jnp.dot(es, w1s_ref[...], preferred_element_type=jnp.float32)
        h = h + jnp.dot(el, w1l_ref[...], preferred_element_type=jnp.float32)
        h = h + jnp.dot(er, w1r_ref[...], preferred_element_type=jnp.float32)
        h = jnp.maximum(h + b1_ref[...], 0.0)
        h = jnp.maximum(
            jnp.dot(h, w2_ref[...], preferred_element_type=jnp.float32) + b2_ref[...], 0.0)
        out_ref[...] = jnp.dot(h, w3_ref[...], preferred_element_type=jnp.float32) + b3_ref[...]

    def full(shape):
        return pl.BlockSpec(shape, lambda: (0,) * len(shape))

    return pl.pallas_call(
        tc_body,
        in_specs=[full((8, D))] * 3 + [full((1, D))] * 3
        + [full((D, D)), full((D, D)), full((D, D)), full((1, D)),
           full((D, D)), full((1, D)), full((D, D)), full((1, D))],
        out_specs=full((8, D)),
        out_shape=jax.ShapeDtypeStruct((8, D), jnp.float32),
    )(s_sum, l_sum, r_sum, bs, bl, br, w1s, w1l, w1r, b1, w2, b2, w3, b3)


def _encoder(x, ei, layers, eye):
    idt = ei.dtype
    loop = jnp.arange(N, dtype=idt)
    padn = ES_PAD - ES
    src = jnp.concatenate([ei[0], loop, jnp.zeros((padn,), idt)])
    dst = jnp.concatenate([ei[1], loop, jnp.full((padn,), NP - 1, idt)])
    xp = jnp.pad(x, ((0, NP - N), (0, 0)))
    xl, xr = _tc_first(xp, layers[0]['Wl'], layers[0]['Wr'])
    num = dens = None
    for i in range(len(layers)):
        num, dens = _sc_layer(xl, xr, src, dst, layers[i]['att'])
        num = num.reshape(NC, NP, D)
        if i < len(layers) - 1:
            xl, xr = _tc_mid(num, dens, eye, layers[i]['bias'].reshape(1, D),
                             layers[i + 1]['Wl'], layers[i + 1]['Wr'])
    return _tc_tail(num, dens, eye)


def kernel(lhs_x, rhs_x, sketch_x, lhs_edge_index, rhs_edge_index,
           sketch_edge_index, params):
    eye = jnp.eye(BLK, dtype=jnp.float32)
    s_sum = _encoder(sketch_x, sketch_edge_index, params['sketch'], eye)
    l_sum = _encoder(lhs_x, lhs_edge_index, params['lhs'], eye)
    r_sum = _encoder(rhs_x, rhs_edge_index, params['rhs'], eye)
    bb = params['backbone']
    w1 = bb['W1']
    out8 = _tc_head(
        s_sum, l_sum, r_sum,
        params['sketch'][-1]['bias'].reshape(1, D),
        params['lhs'][-1]['bias'].reshape(1, D),
        params['rhs'][-1]['bias'].reshape(1, D),
        w1[0:D], w1[D:2 * D], w1[2 * D:3 * D], bb['b1'].reshape(1, D),
        bb['W2'], bb['b2'].reshape(1, D), bb['W3'], bb['b3'].reshape(1, D))
    return out8[0:1]


# speedup vs baseline: 1.6770x; 1.6770x over previous
import functools

import jax
import jax.numpy as jnp
from jax import lax
from jax.experimental import pallas as pl
from jax.experimental.pallas import tpu as pltpu
from jax.experimental.pallas import tpu_sc as plsc

N = 10000
NP = 10240
D = 128
E_BASE = 160000
ES = E_BASE + N
NC = 2
NS = 16
NW = NC * NS
CHUNK = 48
CH_PER_W = 2 * (-(-ES // (NW * CHUNK * 2)))
EPW = CH_PER_W * CHUNK
ES_PAD = NW * EPW
ROWS_PER_TILE = NP // NS
BLK = 1024


def _sc_layer(xl, xr, src, dst, att):
    mesh = plsc.VectorSubcoreMesh(core_axis_name="c", subcore_axis_name="s")

    @functools.partial(
        pl.kernel,
        out_type=[jax.ShapeDtypeStruct((NC * NP, D), jnp.float32),
                  jax.ShapeDtypeStruct((NW, NP), jnp.float32)],
        mesh=mesh,
        scratch_types=[
            pltpu.VMEM((2, CHUNK), jnp.int32),
            pltpu.VMEM((2, CHUNK), jnp.int32),
            pltpu.VMEM((2, CHUNK, D), jnp.float32),
            pltpu.VMEM((2, CHUNK, D), jnp.float32),
            pltpu.VMEM((CHUNK, D), jnp.float32),
            pltpu.VMEM((16, 32), jnp.float32),
            pltpu.VMEM((D,), jnp.float32),
            pltpu.VMEM((NP,), jnp.float32),
            pltpu.VMEM_SHARED((NP, D), jnp.float32),
            pltpu.SemaphoreType.DMA,
            pltpu.SemaphoreType.DMA,
            pltpu.SemaphoreType.DMA,
            pltpu.SemaphoreType.DMA,
        ],
    )
    def body(xl_hbm, xr_hbm, src_hbm, dst_hbm, att_hbm,
             num_hbm, den_hbm,
             src_v, dst_v, xlr, xrr, aug, red, att_v, den_v,
             acc_sh, sem_l0, sem_r0, sem_l1, sem_r1):
        cid = lax.axis_index("c")
        sid = lax.axis_index("s")
        wid = cid * NS + sid
        pltpu.sync_copy(att_hbm, att_v)
        a_j = [att_v[pl.ds(16 * j, 16)] for j in range(D // 16)]

        lane = lax.iota(jnp.int32, 16)
        zero16 = jnp.zeros((16,), jnp.float32)
        zero16i = jnp.zeros((16,), jnp.int32)

        def zero_xlr(e, carry):
            for j in range(D // 16):
                xlr[0, e, pl.ds(j * 16, 16)] = zero16
                xlr[1, e, pl.ds(j * 16, 16)] = zero16
            return carry
        lax.fori_loop(0, CHUNK, zero_xlr, 0)

        def zero_den(t, carry):
            den_v[pl.ds(t * 16, 16)] = zero16
            return carry
        lax.fori_loop(0, NP // 16, zero_den, 0)

        base_row = sid * ROWS_PER_TILE
        full = ROWS_PER_TILE // CHUNK * CHUNK
        for off in range(0, full, CHUNK):
            pltpu.sync_copy(xlr.at[0], acc_sh.at[pl.ds(base_row + off, CHUNK)])
        rem = ROWS_PER_TILE - full
        if rem:
            pltpu.sync_copy(xlr.at[0, pl.ds(0, rem)],
                            acc_sh.at[pl.ds(base_row + full, rem)])
        plsc.subcore_barrier()

        sems = ((sem_l0, sem_r0), (sem_l1, sem_r1))
        cbase = wid * CH_PER_W

        def issue(b, ci):
            ebase = (cbase + ci) * CHUNK
            pltpu.sync_copy(src_hbm.at[pl.ds(ebase, CHUNK)], src_v.at[b])
            pltpu.sync_copy(dst_hbm.at[pl.ds(ebase, CHUNK)], dst_v.at[b])
            pltpu.async_copy(xl_hbm.at[src_v.at[b]], xlr.at[b], sems[b][0])
            pltpu.async_copy(xr_hbm.at[dst_v.at[b]], xrr.at[b], sems[b][1])

        def wait(b):
            pltpu.make_async_copy(xl_hbm.at[src_v.at[b]], xlr.at[b],
                                  sems[b][0]).wait()
            pltpu.make_async_copy(xr_hbm.at[dst_v.at[b]], xrr.at[b],
                                  sems[b][1]).wait()

        def compute(b):
            def group_body(g, carry2):
                dv16 = dst_v[b, pl.ds(g * 16, 16)]
                for l in range(16):
                    e = g * 16 + l
                    acc0 = zero16
                    acc1 = zero16
                    xs = []
                    for j in range(D // 16):
                        vl = xlr[b, e, pl.ds(16 * j, 16)]
                        vr = xrr[b, e, pl.ds(16 * j, 16)]
                        xs.append(vl)
                        z = vl + vr
                        z = jnp.maximum(z, 0.2 * z)
                        if j % 2 == 0:
                            acc0 = acc0 + z * a_j[j]
                        else:
                            acc1 = acc1 + z * a_j[j]
                    acc = acc0 + acc1
                    for sh in (8, 4, 2, 1):
                        red[l, pl.ds(0, 16)] = acc
                        red[l, pl.ds(16, 16)] = acc
                        acc = acc + red[l, pl.ds(sh, 16)]
                    pv = jnp.exp(acc)
                    for j in range(D // 16):
                        aug[e, pl.ds(16 * j, 16)] = xs[j] * pv
                    dd = dv16[l]
                    dbase = (dd >> 4) << 4
                    m = dd & 15
                    diff = jnp.abs(lane - (zero16i + m))
                    oh = jnp.maximum(1 - diff, 0).astype(jnp.float32)
                    den_v[pl.ds(dbase, 16)] = (
                        den_v[pl.ds(dbase, 16)] + pv * oh)
                return carry2
            lax.fori_loop(0, CHUNK // 16, group_body, 0)
            pltpu.sync_copy(aug, acc_sh.at[dst_v.at[b]], add=True)

        last = CH_PER_W - 1
        issue(0, 0)

        def pair_body(i, carry):
            c0 = 2 * i
            issue(1, jnp.minimum(c0 + 1, last))
            wait(0)
            compute(0)
            issue(0, jnp.minimum(c0 + 2, last))
            wait(1)
            compute(1)
            return carry
        lax.fori_loop(0, CH_PER_W // 2, pair_body, 0)
        wait(0)

        plsc.subcore_barrier()
        pltpu.sync_copy(acc_sh.at[pl.ds(base_row, ROWS_PER_TILE)],
                        num_hbm.at[pl.ds(cid * NP + base_row, ROWS_PER_TILE)])
        pltpu.sync_copy(den_v, den_hbm.at[wid])

    return body(xl, xr, src, dst, att)


def _tc_first(x, wl, wr):
    def tc_body(x_ref, wl_ref, wr_ref, xl_ref, xr_ref):
        h = x_ref[...]
        xl_ref[...] = jnp.dot(h, wl_ref[...], preferred_element_type=jnp.float32)
        xr_ref[...] = jnp.dot(h, wr_ref[...], preferred_element_type=jnp.float32)

    return pl.pallas_call(
        tc_body,
        grid=(NP // BLK,),
        in_specs=[pl.BlockSpec((BLK, D), lambda i: (i, 0)),
                  pl.BlockSpec((D, D), lambda i: (0, 0)),
                  pl.BlockSpec((D, D), lambda i: (0, 0))],
        out_specs=[pl.BlockSpec((BLK, D), lambda i: (i, 0)),
                   pl.BlockSpec((BLK, D), lambda i: (i, 0))],
        out_shape=[jax.ShapeDtypeStruct((NP, D), jnp.float32)] * 2,
    )(x, wl, wr)


def _den_col(dens_blk, eye_blk):
    s = jnp.sum(dens_blk, axis=0, keepdims=True)
    sb = jnp.broadcast_to(s, (8, BLK))
    mm = lax.dot_general(eye_blk, sb, (((1,), (1,)), ((), ())),
                         preferred_element_type=jnp.float32)
    return mm[:, 0:1]


def _tc_mid(num, dens, eye, bias, wl, wr):
    def tc_body(num_ref, den_ref, eye_ref, b_ref, wl_ref, wr_ref,
                xl_ref, xr_ref):
        s = num_ref[0] + num_ref[1]
        dcol = _den_col(den_ref[...], eye_ref[...])
        h = s / (dcol + 1e-16) + b_ref[...]
        h = jnp.maximum(h, 0.0)
        xl_ref[...] = jnp.dot(h, wl_ref[...], preferred_element_type=jnp.float32)
        xr_ref[...] = jnp.dot(h, wr_ref[...], preferred_element_type=jnp.float32)

    return pl.pallas_call(
        tc_body,
        grid=(NP // BLK,),
        in_specs=[pl.BlockSpec((2, BLK, D), lambda i: (0, i, 0)),
                  pl.BlockSpec((NW, BLK), lambda i: (0, i)),
                  pl.BlockSpec((BLK, BLK), lambda i: (0, 0)),
                  pl.BlockSpec((1, D), lambda i: (0, 0)),
                  pl.BlockSpec((D, D), lambda i: (0, 0)),
                  pl.BlockSpec((D, D), lambda i: (0, 0))],
        out_specs=[pl.BlockSpec((BLK, D), lambda i: (i, 0)),
                   pl.BlockSpec((BLK, D), lambda i: (i, 0))],
        out_shape=[jax.ShapeDtypeStruct((NP, D), jnp.float32)] * 2,
    )(num, dens, eye, bias, wl, wr)


def _tc_tail(num, dens, eye):
    def tc_body(num_ref, den_ref, eye_ref, out_ref):
        i = pl.program_id(0)
        s = num_ref[0] + num_ref[1]
        dcol = _den_col(den_ref[...], eye_ref[...])
        h = s / (dcol + 1e-16)
        rows = i * BLK + lax.broadcasted_iota(jnp.int32, (BLK, 1), 0)
        h = jnp.where(rows < N, h, 0.0)
        part = jnp.sum(h, axis=0, keepdims=True)

        @pl.when(i == 0)
        def _():
            out_ref[...] = jnp.zeros_like(out_ref)

        out_ref[0:1, :] += part

    return pl.pallas_call(
        tc_body,
        grid=(NP // BLK,),
        in_specs=[pl.BlockSpec((2, BLK, D), lambda i: (0, i, 0)),
                  pl.BlockSpec((NW, BLK), lambda i: (0, i)),
                  pl.BlockSpec((BLK, BLK), lambda i: (0, 0))],
        out_specs=pl.BlockSpec((8, D), lambda i: (0, 0)),
        out_shape=jax.ShapeDtypeStruct((8, D), jnp.float32),
    )(num, dens, eye)


def _tc_head(s_sum, l_sum, r_sum, bs, bl, br, w1s, w1l, w1r, b1, w2, b2, w3, b3):
    inv_n = 1.0 / N

    def tc_body(ss_ref, sl_ref, sr_ref, bs_ref, bl_ref, br_ref,
                w1s_ref, w1l_ref, w1r_ref, b1_ref, w2_ref, b2_ref,
                w3_ref, b3_ref, out_ref):
        es = ss_ref[...] * inv_n + bs_ref[...]
        el = sl_ref[...] * inv_n + bl_ref[...]
        er = sr_ref[...] * inv_n + br_ref[...]
        h = jnp.dot(es, w1s_ref[...], preferred_element_type=jnp.float32)
        h = h + jnp.dot(el, w1l_ref[...], preferred_element_type=jnp.float32)
        h = h + jnp.dot(er, w1r_ref[...], preferred_element_type=jnp.float32)
        h = jnp.maximum(h + b1_ref[...], 0.0)
        h = jnp.maximum(
            jnp.dot(h, w2_ref[...], preferred_element_type=jnp.float32) + b2_ref[...], 0.0)
        out_ref[...] = jnp.dot(h, w3_ref[...], preferred_element_type=jnp.float32) + b3_ref[...]

    def full(shape):
        return pl.BlockSpec(shape, lambda: (0,) * len(shape))

    return pl.pallas_call(
        tc_body,
        in_specs=[full((8, D))] * 3 + [full((1, D))] * 3
        + [full((D, D)), full((D, D)), full((D, D)), full((1, D)),
           full((D, D)), full((1, D)), full((D, D)), full((1, D))],
        out_specs=full((8, D)),
        out_shape=jax.ShapeDtypeStruct((8, D), jnp.float32),
    )(s_sum, l_sum, r_sum, bs, bl, br, w1s, w1l, w1r, b1, w2, b2, w3, b3)


def _encoder(x, ei, layers, eye):
    idt = ei.dtype
    loop = jnp.arange(N, dtype=idt)
    padn = ES_PAD - ES
    src = jnp.concatenate([ei[0], loop, jnp.zeros((padn,), idt)])
    dst = jnp.concatenate([ei[1], loop, jnp.full((padn,), NP - 1, idt)])
    xp = jnp.pad(x, ((0, NP - N), (0, 0)))
    xl, xr = _tc_first(xp, layers[0]['Wl'], layers[0]['Wr'])
    num = dens = None
    for i in range(len(layers)):
        num, dens = _sc_layer(xl, xr, src, dst, layers[i]['att'])
        num = num.reshape(NC, NP, D)
        if i < len(layers) - 1:
            xl, xr = _tc_mid(num, dens, eye, layers[i]['bias'].reshape(1, D),
                             layers[i + 1]['Wl'], layers[i + 1]['Wr'])
    return _tc_tail(num, dens, eye)


def kernel(lhs_x, rhs_x, sketch_x, lhs_edge_index, rhs_edge_index,
           sketch_edge_index, params):
    eye = jnp.eye(BLK, dtype=jnp.float32)
    s_sum = _encoder(sketch_x, sketch_edge_index, params['sketch'], eye)
    l_sum = _encoder(lhs_x, lhs_edge_index, params['lhs'], eye)
    r_sum = _encoder(rhs_x, rhs_edge_index, params['rhs'], eye)
    bb = params['backbone']
    w1 = bb['W1']
    out8 = _tc_head(
        s_sum, l_sum, r_sum,
        params['sketch'][-1]['bias'].reshape(1, D),
        params['lhs'][-1]['bias'].reshape(1, D),
        params['rhs'][-1]['bias'].reshape(1, D),
        w1[0:D], w1[D:2 * D], w1[2 * D:3 * D], bb['b1'].reshape(1, D),
        bb['W2'], bb['b2'].reshape(1, D), bb['W3'], bb['b3'].reshape(1, D))
    return out8[0:1]


# speedup vs baseline: 1.9677x; 1.1733x over previous
import functools

import jax
import jax.numpy as jnp
from jax import lax
from jax.experimental import pallas as pl
from jax.experimental.pallas import tpu as pltpu
from jax.experimental.pallas import tpu_sc as plsc

N = 10000
NP = 10240
D = 128
E_BASE = 160000
ES = E_BASE + N
NC = 2
NS = 16
NW = NC * NS
CHUNK = 64
CH_PER_W = 2 * (-(-ES // (NW * CHUNK * 2)))
EPW = CH_PER_W * CHUNK
ES_PAD = NW * EPW
ROWS_PER_TILE = NP // NS
BLK = 1024


def _sc_layer(xl, xr, src, dst, att):
    mesh = plsc.VectorSubcoreMesh(core_axis_name="c", subcore_axis_name="s")

    @functools.partial(
        pl.kernel,
        out_type=[jax.ShapeDtypeStruct((NC * NP, D), jnp.float32),
                  jax.ShapeDtypeStruct((NW, NP), jnp.float32)],
        mesh=mesh,
        scratch_types=[
            pltpu.VMEM((2, CHUNK), jnp.int32),
            pltpu.VMEM((2, CHUNK), jnp.int32),
            pltpu.VMEM((2, CHUNK, D), jnp.float32),
            pltpu.VMEM((2, CHUNK, D), jnp.float32),
            pltpu.VMEM((16, 32), jnp.float32),
            pltpu.VMEM((D,), jnp.float32),
            pltpu.VMEM((NP,), jnp.float32),
            pltpu.VMEM_SHARED((NP, D), jnp.float32),
            pltpu.SemaphoreType.DMA,
            pltpu.SemaphoreType.DMA,
            pltpu.SemaphoreType.DMA,
            pltpu.SemaphoreType.DMA,
            pltpu.SemaphoreType.DMA,
            pltpu.SemaphoreType.DMA,
        ],
    )
    def body(xl_hbm, xr_hbm, src_hbm, dst_hbm, att_hbm,
             num_hbm, den_hbm,
             src_v, dst_v, xlr, xrr, red, att_v, den_v,
             acc_sh, sem_l0, sem_r0, sem_l1, sem_r1, sem_i0, sem_i1):
        cid = lax.axis_index("c")
        sid = lax.axis_index("s")
        wid = cid * NS + sid
        pltpu.sync_copy(att_hbm, att_v)
        a_j = [att_v[pl.ds(16 * j, 16)] for j in range(D // 16)]

        lane = lax.iota(jnp.int32, 16)
        zero16 = jnp.zeros((16,), jnp.float32)
        zero16i = jnp.zeros((16,), jnp.int32)

        def zero_xlr(e, carry):
            for j in range(D // 16):
                xlr[0, e, pl.ds(j * 16, 16)] = zero16
                xlr[1, e, pl.ds(j * 16, 16)] = zero16
            return carry
        lax.fori_loop(0, CHUNK, zero_xlr, 0)

        def zero_den(t, carry):
            den_v[pl.ds(t * 16, 16)] = zero16
            return carry
        lax.fori_loop(0, NP // 16, zero_den, 0)

        base_row = sid * ROWS_PER_TILE
        full = ROWS_PER_TILE // CHUNK * CHUNK
        for off in range(0, full, CHUNK):
            pltpu.sync_copy(xlr.at[0], acc_sh.at[pl.ds(base_row + off, CHUNK)])
        rem = ROWS_PER_TILE - full
        if rem:
            pltpu.sync_copy(xlr.at[0, pl.ds(0, rem)],
                            acc_sh.at[pl.ds(base_row + full, rem)])
        plsc.subcore_barrier()

        sems = ((sem_l0, sem_r0), (sem_l1, sem_r1))
        isems = (sem_i0, sem_i1)
        cbase = wid * CH_PER_W

        def idx_copy(b, ci):
            ebase = (cbase + ci) * CHUNK
            pltpu.async_copy(src_hbm.at[pl.ds(ebase, CHUNK)], src_v.at[b],
                             isems[b])
            pltpu.async_copy(dst_hbm.at[pl.ds(ebase, CHUNK)], dst_v.at[b],
                             isems[b])

        def idx_wait(b, ci):
            ebase = (cbase + ci) * CHUNK
            pltpu.make_async_copy(src_hbm.at[pl.ds(ebase, CHUNK)],
                                  src_v.at[b], isems[b]).wait()
            pltpu.make_async_copy(dst_hbm.at[pl.ds(ebase, CHUNK)],
                                  dst_v.at[b], isems[b]).wait()

        def gathers(b):
            pltpu.async_copy(xl_hbm.at[src_v.at[b]], xlr.at[b], sems[b][0])
            pltpu.async_copy(xr_hbm.at[dst_v.at[b]], xrr.at[b], sems[b][1])

        def wait(b):
            pltpu.make_async_copy(xl_hbm.at[src_v.at[b]], xlr.at[b],
                                  sems[b][0]).wait()
            pltpu.make_async_copy(xr_hbm.at[dst_v.at[b]], xrr.at[b],
                                  sems[b][1]).wait()

        def compute(b):
            def group_body(g, carry2):
                dv16 = dst_v[b, pl.ds(g * 16, 16)]
                for l in range(16):
                    e = g * 16 + l
                    acc0 = zero16
                    acc1 = zero16
                    xs = []
                    for j in range(D // 16):
                        vl = xlr[b, e, pl.ds(16 * j, 16)]
                        vr = xrr[b, e, pl.ds(16 * j, 16)]
                        xs.append(vl)
                        z = vl + vr
                        z = jnp.maximum(z, 0.2 * z)
                        if j % 2 == 0:
                            acc0 = acc0 + z * a_j[j]
                        else:
                            acc1 = acc1 + z * a_j[j]
                    acc = acc0 + acc1
                    for sh in (8, 4, 2, 1):
                        red[l, pl.ds(0, 16)] = acc
                        red[l, pl.ds(16, 16)] = acc
                        acc = acc + red[l, pl.ds(sh, 16)]
                    pv = jnp.exp(acc)
                    for j in range(D // 16):
                        xlr[b, e, pl.ds(16 * j, 16)] = xs[j] * pv
                    dd = dv16[l]
                    dbase = (dd >> 4) << 4
                    m = dd & 15
                    diff = jnp.abs(lane - (zero16i + m))
                    oh = jnp.maximum(1 - diff, 0).astype(jnp.float32)
                    den_v[pl.ds(dbase, 16)] = (
                        den_v[pl.ds(dbase, 16)] + pv * oh)
                return carry2
            lax.fori_loop(0, CHUNK // 16, group_body, 0)
            pltpu.sync_copy(xlr.at[b], acc_sh.at[dst_v.at[b]], add=True)

        last = CH_PER_W - 1
        idx_copy(0, 0)
        idx_copy(1, 1)
        idx_wait(0, 0)
        gathers(0)

        def pair_body(i, carry):
            c0 = 2 * i
            idx_wait(1, jnp.minimum(c0 + 1, last))
            gathers(1)
            wait(0)
            compute(0)
            idx_copy(0, jnp.minimum(c0 + 2, last))
            idx_wait(0, jnp.minimum(c0 + 2, last))
            gathers(0)
            wait(1)
            compute(1)
            idx_copy(1, jnp.minimum(c0 + 3, last))
            return carry
        lax.fori_loop(0, CH_PER_W // 2, pair_body, 0)
        wait(0)
        idx_wait(1, last)

        plsc.subcore_barrier()
        pltpu.sync_copy(acc_sh.at[pl.ds(base_row, ROWS_PER_TILE)],
                        num_hbm.at[pl.ds(cid * NP + base_row, ROWS_PER_TILE)])
        pltpu.sync_copy(den_v, den_hbm.at[wid])

    return body(xl, xr, src, dst, att)


def _tc_first(x, wl, wr):
    def tc_body(x_ref, wl_ref, wr_ref, xl_ref, xr_ref):
        h = x_ref[...]
        xl_ref[...] = jnp.dot(h, wl_ref[...], preferred_element_type=jnp.float32)
        xr_ref[...] = jnp.dot(h, wr_ref[...], preferred_element_type=jnp.float32)

    return pl.pallas_call(
        tc_body,
        grid=(NP // BLK,),
        in_specs=[pl.BlockSpec((BLK, D), lambda i: (i, 0)),
                  pl.BlockSpec((D, D), lambda i: (0, 0)),
                  pl.BlockSpec((D, D), lambda i: (0, 0))],
        out_specs=[pl.BlockSpec((BLK, D), lambda i: (i, 0)),
                   pl.BlockSpec((BLK, D), lambda i: (i, 0))],
        out_shape=[jax.ShapeDtypeStruct((NP, D), jnp.float32)] * 2,
    )(x, wl, wr)


def _den_col(dens_blk, eye_blk):
    s = jnp.sum(dens_blk, axis=0, keepdims=True)
    sb = jnp.broadcast_to(s, (8, BLK))
    mm = lax.dot_general(eye_blk, sb, (((1,), (1,)), ((), ())),
                         preferred_element_type=jnp.float32)
    return mm[:, 0:1]


def _tc_mid(num, dens, eye, bias, wl, wr):
    def tc_body(num_ref, den_ref, eye_ref, b_ref, wl_ref, wr_ref,
                xl_ref, xr_ref):
        s = num_ref[0] + num_ref[1]
        dcol = _den_col(den_ref[...], eye_ref[...])
        h = s / (dcol + 1e-16) + b_ref[...]
        h = jnp.maximum(h, 0.0)
        xl_ref[...] = jnp.dot(h, wl_ref[...], preferred_element_type=jnp.float32)
        xr_ref[...] = jnp.dot(h, wr_ref[...], preferred_element_type=jnp.float32)

    return pl.pallas_call(
        tc_body,
        grid=(NP // BLK,),
        in_specs=[pl.BlockSpec((2, BLK, D), lambda i: (0, i, 0)),
                  pl.BlockSpec((NW, BLK), lambda i: (0, i)),
                  pl.BlockSpec((BLK, BLK), lambda i: (0, 0)),
                  pl.BlockSpec((1, D), lambda i: (0, 0)),
                  pl.BlockSpec((D, D), lambda i: (0, 0)),
                  pl.BlockSpec((D, D), lambda i: (0, 0))],
        out_specs=[pl.BlockSpec((BLK, D), lambda i: (i, 0)),
                   pl.BlockSpec((BLK, D), lambda i: (i, 0))],
        out_shape=[jax.ShapeDtypeStruct((NP, D), jnp.float32)] * 2,
    )(num, dens, eye, bias, wl, wr)


def _tc_tail(num, dens, eye):
    def tc_body(num_ref, den_ref, eye_ref, out_ref):
        i = pl.program_id(0)
        s = num_ref[0] + num_ref[1]
        dcol = _den_col(den_ref[...], eye_ref[...])
        h = s / (dcol + 1e-16)
        rows = i * BLK + lax.broadcasted_iota(jnp.int32, (BLK, 1), 0)
        h = jnp.where(rows < N, h, 0.0)
        part = jnp.sum(h, axis=0, keepdims=True)

        @pl.when(i == 0)
        def _():
            out_ref[...] = jnp.zeros_like(out_ref)

        out_ref[0:1, :] += part

    return pl.pallas_call(
        tc_body,
        grid=(NP // BLK,),
        in_specs=[pl.BlockSpec((2, BLK, D), lambda i: (0, i, 0)),
                  pl.BlockSpec((NW, BLK), lambda i: (0, i)),
                  pl.BlockSpec((BLK, BLK), lambda i: (0, 0))],
        out_specs=pl.BlockSpec((8, D), lambda i: (0, 0)),
        out_shape=jax.ShapeDtypeStruct((8, D), jnp.float32),
    )(num, dens, eye)


def _tc_head(s_sum, l_sum, r_sum, bs, bl, br, w1s, w1l, w1r, b1, w2, b2, w3, b3):
    inv_n = 1.0 / N

    def tc_body(ss_ref, sl_ref, sr_ref, bs_ref, bl_ref, br_ref,
                w1s_ref, w1l_ref, w1r_ref, b1_ref, w2_ref, b2_ref,
                w3_ref, b3_ref, out_ref):
        es = ss_ref[...] * inv_n + bs_ref[...]
        el = sl_ref[...] * inv_n + bl_ref[...]
        er = sr_ref[...] * inv_n + br_ref[...]
        h = jnp.dot(es, w1s_ref[...], preferred_element_type=jnp.float32)
        h = h + jnp.dot(el, w1l_ref[...], preferred_element_type=jnp.float32)
        h = h + jnp.dot(er, w1r_ref[...], preferred_element_type=jnp.float32)
        h = jnp.maximum(h + b1_ref[...], 0.0)
        h = jnp.maximum(
            jnp.dot(h, w2_ref[...], preferred_element_type=jnp.float32) + b2_ref[...], 0.0)
        out_ref[...] = jnp.dot(h, w3_ref[...], preferred_element_type=jnp.float32) + b3_ref[...]

    def full(shape):
        return pl.BlockSpec(shape, lambda: (0,) * len(shape))

    return pl.pallas_call(
        tc_body,
        in_specs=[full((8, D))] * 3 + [full((1, D))] * 3
        + [full((D, D)), full((D, D)), full((D, D)), full((1, D)),
           full((D, D)), full((1, D)), full((D, D)), full((1, D))],
        out_specs=full((8, D)),
        out_shape=jax.ShapeDtypeStruct((8, D), jnp.float32),
    )(s_sum, l_sum, r_sum, bs, bl, br, w1s, w1l, w1r, b1, w2, b2, w3, b3)


def _encoder(x, ei, layers, eye):
    idt = ei.dtype
    loop = jnp.arange(N, dtype=idt)
    padn = ES_PAD - ES
    src = jnp.concatenate([ei[0], loop, jnp.zeros((padn,), idt)])
    dst = jnp.concatenate([ei[1], loop, jnp.full((padn,), NP - 1, idt)])
    xp = jnp.pad(x, ((0, NP - N), (0, 0)))
    xl, xr = _tc_first(xp, layers[0]['Wl'], layers[0]['Wr'])
    num = dens = None
    for i in range(len(layers)):
        num, dens = _sc_layer(xl, xr, src, dst, layers[i]['att'])
        num = num.reshape(NC, NP, D)
        if i < len(layers) - 1:
            xl, xr = _tc_mid(num, dens, eye, layers[i]['bias'].reshape(1, D),
                             layers[i + 1]['Wl'], layers[i + 1]['Wr'])
    return _tc_tail(num, dens, eye)


def kernel(lhs_x, rhs_x, sketch_x, lhs_edge_index, rhs_edge_index,
           sketch_edge_index, params):
    eye = jnp.eye(BLK, dtype=jnp.float32)
    s_sum = _encoder(sketch_x, sketch_edge_index, params['sketch'], eye)
    l_sum = _encoder(lhs_x, lhs_edge_index, params['lhs'], eye)
    r_sum = _encoder(rhs_x, rhs_edge_index, params['rhs'], eye)
    bb = params['backbone']
    w1 = bb['W1']
    out8 = _tc_head(
        s_sum, l_sum, r_sum,
        params['sketch'][-1]['bias'].reshape(1, D),
        params['lhs'][-1]['bias'].reshape(1, D),
        params['rhs'][-1]['bias'].reshape(1, D),
        w1[0:D], w1[D:2 * D], w1[2 * D:3 * D], bb['b1'].reshape(1, D),
        bb['W2'], bb['b2'].reshape(1, D), bb['W3'], bb['b3'].reshape(1, D))
    return out8[0:1]
